# baseline (speedup 1.0000x reference)
import jax
import jax.numpy as jnp
from jax.experimental import pallas as pl
from jax.experimental.pallas import tpu as pltpu

_INTERPRET = False


def _warp_kernel(flows_ref, img_ref, imgflat_ref, out_ref):
    f32 = jnp.float32
    L = flows_ref.shape[1]
    CW, H = img_ref.shape[2], img_ref.shape[3]
    C = out_ref.shape[2]
    W = CW // C
    HW = H * W
    t = pl.program_id(1)

    pi = jax.lax.broadcasted_iota(jnp.int32, (1, HW), 1)
    jf = (pi % W).astype(f32)
    i_f = (pi // W).astype(f32)
    base_x = (2.0 * jf + 1.0) * (1.0 / W) - 1.0
    base_y = (2.0 * i_f + 1.0) * (1.0 / H) - 1.0
    iota64 = jax.lax.broadcasted_iota(jnp.int32, (H, HW), 0).astype(f32)
    ecol = jax.lax.broadcasted_iota(jnp.int32, (C, CW), 0)
    erow = jax.lax.broadcasted_iota(jnp.int32, (C, CW), 1) // W
    E8 = jnp.where(ecol == erow, 1.0, 0.0).astype(jnp.bfloat16)

    bf16 = jnp.bfloat16

    def build_masks(dx, dy):
        a = base_x + dx + 1.0
        gx = a - 2.0 * jnp.floor(a * 0.5) - 1.0
        ix = (gx + 1.0) * (0.5 * W) - 0.5
        iy = (base_y + dy + 1.0) * (0.5 * H) - 0.5
        MyT = jnp.maximum(0.0, 1.0 - jnp.abs(iota64 - iy)).astype(bf16)
        MxT = jnp.maximum(0.0, 1.0 - jnp.abs(iota64 - ix)).astype(bf16)
        return MyT, MxT

    NC = 2
    PC = HW // NC

    def warp(k, MyT, MxT):
        imgk = img_ref[0, k]
        outs = []
        for ci in range(NC):
            sl = slice(ci * PC, (ci + 1) * PC)
            RTc = jnp.dot(imgk, MyT[:, sl], preferred_element_type=f32)
            Vbc = (RTc.astype(bf16).reshape(C, W, PC)
                   * MxT[:, sl][None]).reshape(CW, PC)
            outs.append(jnp.dot(E8, Vbc, preferred_element_type=f32))
        return jnp.concatenate(outs, axis=1)

    def body(it, carry):
        dx, dy, acc = carry
        k = t - it
        MyT, MxT = build_masks(dx, dy)
        outk = warp(k, MyT, MxT)
        dx2 = dx + flows_ref[0, k, 0]
        dy2 = dy + flows_ref[0, k, 1]
        return (dx2, dy2, acc + outk)

    z = jnp.zeros((1, HW), f32)
    acc0 = imgflat_ref[0, 0].astype(f32)
    dx0 = z + flows_ref[0, t, 0]
    dy0 = z + flows_ref[0, t, 1]
    _, _, acc = jax.lax.fori_loop(1, t + 1, body, (dx0, dy0, acc0))
    out_ref[0, 0] = acc


@jax.jit
def kernel(flows, images):
    B, L, C, H, W = images.shape
    imgT = images.transpose(0, 1, 2, 4, 3).reshape(B, L, C * W, H).astype(jnp.bfloat16)
    imgflat = images.reshape(B, L, C, H * W).astype(jnp.float32)
    flows_r = flows.astype(jnp.float32).reshape(B, L, 2, 1, H * W)
    out = pl.pallas_call(
        _warp_kernel,
        grid=(B, L),
        in_specs=[
            pl.BlockSpec((1, L, 2, 1, H * W), lambda b, t: (b, 0, 0, 0, 0)),
            pl.BlockSpec((1, L, C * W, H), lambda b, t: (b, 0, 0, 0)),
            pl.BlockSpec((1, 1, C, H * W), lambda b, t: (b, t, 0, 0)),
        ],
        out_specs=pl.BlockSpec((1, 1, C, H * W), lambda b, t: (b, t, 0, 0)),
        out_shape=jax.ShapeDtypeStruct((B, L, C, H * W), jnp.float32),
        compiler_params=pltpu.CompilerParams(
            dimension_semantics=("parallel", "arbitrary"),
        ),
        interpret=_INTERPRET,
    )(flows_r, imgT, imgflat)
    return out.reshape(B, L, C, H, W)


# speedup vs baseline: 1.1553x; 1.1553x over previous
import jax
import jax.numpy as jnp
from jax.experimental import pallas as pl
from jax.experimental.pallas import tpu as pltpu

_INTERPRET = False


def _warp_kernel(flows_ref, img_ref, imgflat_ref, out_ref):
    f32 = jnp.float32
    L = flows_ref.shape[1]
    CW, H = img_ref.shape[2], img_ref.shape[3]
    C = out_ref.shape[2]
    W = CW // C
    HW = H * W
    t = pl.program_id(1)

    pi = jax.lax.broadcasted_iota(jnp.int32, (1, HW), 1)
    jf = (pi % W).astype(f32)
    i_f = (pi // W).astype(f32)
    base_x = (2.0 * jf + 1.0) * (1.0 / W) - 1.0
    base_y = (2.0 * i_f + 1.0) * (1.0 / H) - 1.0
    iota64 = jax.lax.broadcasted_iota(jnp.int32, (H, HW), 0).astype(f32)
    ecol = jax.lax.broadcasted_iota(jnp.int32, (C, CW), 0)
    erow = jax.lax.broadcasted_iota(jnp.int32, (C, CW), 1) // W
    E8 = jnp.where(ecol == erow, 1.0, 0.0).astype(jnp.bfloat16)

    bf16 = jnp.bfloat16

    def build_masks(dx, dy):
        a = base_x + dx + 1.0
        gx = a - 2.0 * jnp.floor(a * 0.5) - 1.0
        ix = (gx + 1.0) * (0.5 * W) - 0.5
        iy = (base_y + dy + 1.0) * (0.5 * H) - 0.5
        MyT = jnp.maximum(0.0, 1.0 - jnp.abs(iota64 - iy)).astype(bf16)
        MxT = jnp.maximum(0.0, 1.0 - jnp.abs(iota64 - ix)).astype(bf16)
        return MyT, MxT

    def vmask(k, MyT, MxT):
        imgk = img_ref[0, k]
        RT = jnp.dot(imgk, MyT, preferred_element_type=f32)
        return (RT.astype(bf16).reshape(C, W, HW)
                * MxT[None]).reshape(CW, HW)

    def warp(k, MyT, MxT):
        return jnp.dot(E8, vmask(k, MyT, MxT),
                       preferred_element_type=f32)

    def body2(s, carry):
        dx, dy, acc = carry
        k1 = t - 1 - 2 * s
        MyA, MxA = build_masks(dx, dy)
        dxb = dx + flows_ref[0, k1, 0]
        dyb = dy + flows_ref[0, k1, 1]
        MyB, MxB = build_masks(dxb, dyb)
        VbA = vmask(k1, MyA, MxA)
        VbB = vmask(k1 - 1, MyB, MxB)
        outAB = jnp.dot(E8, VbA + VbB, preferred_element_type=f32)
        dx2 = dxb + flows_ref[0, k1 - 1, 0]
        dy2 = dyb + flows_ref[0, k1 - 1, 1]
        return (dx2, dy2, acc + outAB)

    z = jnp.zeros((1, HW), f32)
    acc0 = imgflat_ref[0, 0].astype(f32)
    dx0 = z + flows_ref[0, t, 0]
    dy0 = z + flows_ref[0, t, 1]
    dx, dy, acc = jax.lax.fori_loop(0, t // 2, body2, (dx0, dy0, acc0))

    @pl.when(t % 2 == 1)
    def _():
        MyE, MxE = build_masks(dx, dy)
        out_ref[0, 0] = acc + warp(0, MyE, MxE)

    @pl.when(t % 2 == 0)
    def _():
        out_ref[0, 0] = acc


@jax.jit
def kernel(flows, images):
    B, L, C, H, W = images.shape
    imgT = images.transpose(0, 1, 2, 4, 3).reshape(B, L, C * W, H).astype(jnp.bfloat16)
    imgflat = images.reshape(B, L, C, H * W).astype(jnp.float32)
    flows_r = flows.astype(jnp.float32).reshape(B, L, 2, 1, H * W)
    out = pl.pallas_call(
        _warp_kernel,
        grid=(B, L),
        in_specs=[
            pl.BlockSpec((1, L, 2, 1, H * W), lambda b, t: (b, 0, 0, 0, 0)),
            pl.BlockSpec((1, L, C * W, H), lambda b, t: (b, 0, 0, 0)),
            pl.BlockSpec((1, 1, C, H * W), lambda b, t: (b, t, 0, 0)),
        ],
        out_specs=pl.BlockSpec((1, 1, C, H * W), lambda b, t: (b, t, 0, 0)),
        out_shape=jax.ShapeDtypeStruct((B, L, C, H * W), jnp.float32),
        compiler_params=pltpu.CompilerParams(
            dimension_semantics=("parallel", "arbitrary"),
        ),
        interpret=_INTERPRET,
    )(flows_r, imgT, imgflat)
    return out.reshape(B, L, C, H, W)


# speedup vs baseline: 1.3215x; 1.1438x over previous
import jax
import jax.numpy as jnp
from jax.experimental import pallas as pl
from jax.experimental.pallas import tpu as pltpu

_INTERPRET = False


def _warp_kernel(flows_ref, img_ref, imgflat_ref, out_ref):
    f32 = jnp.float32
    L = flows_ref.shape[1]
    CW, H = img_ref.shape[2], img_ref.shape[3]
    C = out_ref.shape[2]
    W = CW // C
    HW = H * W
    t = pl.program_id(1)

    pi = jax.lax.broadcasted_iota(jnp.int32, (1, HW), 1)
    jf = (pi % W).astype(f32)
    i_f = (pi // W).astype(f32)
    base_x = (2.0 * jf + 1.0) * (1.0 / W) - 1.0
    base_y = (2.0 * i_f + 1.0) * (1.0 / H) - 1.0
    iota64 = jax.lax.broadcasted_iota(jnp.int32, (H, HW), 0).astype(f32)
    ecol = jax.lax.broadcasted_iota(jnp.int32, (C, CW), 0)
    erow = jax.lax.broadcasted_iota(jnp.int32, (C, CW), 1) // W
    E8 = jnp.where(ecol == erow, 1.0, 0.0).astype(jnp.bfloat16)

    bf16 = jnp.bfloat16

    def build_masks(dx, dy):
        a = base_x + dx + 1.0
        gx = a - 2.0 * jnp.floor(a * 0.5) - 1.0
        ix = (gx + 1.0) * (0.5 * W) - 0.5
        iy = (base_y + dy + 1.0) * (0.5 * H) - 0.5
        MyT = jnp.maximum(0.0, 1.0 - jnp.abs(iota64 - iy)).astype(bf16)
        MxT = jnp.maximum(0.0, 1.0 - jnp.abs(iota64 - ix)).astype(bf16)
        return MyT, MxT

    def vmask(k, MyT, MxT):
        imgk = img_ref[0, k]
        RT = jnp.dot(imgk, MyT, preferred_element_type=f32)
        return (RT.astype(bf16).reshape(C, W, HW)
                * MxT[None]).reshape(CW, HW)

    def warp(k, MyT, MxT):
        return jnp.dot(E8, vmask(k, MyT, MxT),
                       preferred_element_type=f32)

    NU = 4

    def body4(s, carry):
        dx, dy, acc = carry
        ktop = t - 1 - NU * s
        Vsum = None
        for u in range(NU):
            My, Mx = build_masks(dx, dy)
            Vb = vmask(ktop - u, My, Mx)
            Vsum = Vb if Vsum is None else Vsum + Vb
            dx = dx + flows_ref[0, ktop - u, 0]
            dy = dy + flows_ref[0, ktop - u, 1]
        out4 = jnp.dot(E8, Vsum, preferred_element_type=f32)
        return (dx, dy, acc + out4)

    def body1(j, carry):
        dx, dy, acc = carry
        k = t % NU - 1 - j
        My, Mx = build_masks(dx, dy)
        outk = warp(k, My, Mx)
        dx2 = dx + flows_ref[0, k, 0]
        dy2 = dy + flows_ref[0, k, 1]
        return (dx2, dy2, acc + outk)

    z = jnp.zeros((1, HW), f32)
    acc0 = imgflat_ref[0, 0].astype(f32)
    dx0 = z + flows_ref[0, t, 0]
    dy0 = z + flows_ref[0, t, 1]
    carry = jax.lax.fori_loop(0, t // NU, body4, (dx0, dy0, acc0))
    _, _, acc = jax.lax.fori_loop(0, t % NU, body1, carry)
    out_ref[0, 0] = acc


@jax.jit
def kernel(flows, images):
    B, L, C, H, W = images.shape
    imgT = images.transpose(0, 1, 2, 4, 3).reshape(B, L, C * W, H).astype(jnp.bfloat16)
    imgflat = images.reshape(B, L, C, H * W).astype(jnp.float32)
    flows_r = flows.astype(jnp.float32).reshape(B, L, 2, 1, H * W)
    out = pl.pallas_call(
        _warp_kernel,
        grid=(B, L),
        in_specs=[
            pl.BlockSpec((1, L, 2, 1, H * W), lambda b, t: (b, 0, 0, 0, 0)),
            pl.BlockSpec((1, L, C * W, H), lambda b, t: (b, 0, 0, 0)),
            pl.BlockSpec((1, 1, C, H * W), lambda b, t: (b, t, 0, 0)),
        ],
        out_specs=pl.BlockSpec((1, 1, C, H * W), lambda b, t: (b, t, 0, 0)),
        out_shape=jax.ShapeDtypeStruct((B, L, C, H * W), jnp.float32),
        compiler_params=pltpu.CompilerParams(
            dimension_semantics=("parallel", "arbitrary"),
        ),
        interpret=_INTERPRET,
    )(flows_r, imgT, imgflat)
    return out.reshape(B, L, C, H, W)
